# baseline (speedup 1.0000x reference)
import functools

import jax
import jax.numpy as jnp
from jax import lax
from jax.experimental import pallas as pl
from jax.experimental.pallas import tpu as pltpu
from jax.experimental.pallas import tpu_sc as plsc

B, S, D = 16, 512, 1024
NC, NS, L = 2, 16, 16
NW = NC * NS
ROWS = B * S
RPW = ROWS // NW
CHUNK = 64
NCHUNK = RPW // CHUNK


def _make_kernel():
    mesh = plsc.VectorSubcoreMesh(core_axis_name="c", subcore_axis_name="s")

    @functools.partial(
        pl.kernel,
        mesh=mesh,
        out_type=jax.ShapeDtypeStruct((ROWS, D), jnp.float32),
        scratch_types=[
            pltpu.VMEM((CHUNK,), jnp.int32),
            pltpu.VMEM((CHUNK, D), jnp.float32),
            pltpu.SemaphoreType.DMA,
        ],
    )
    def k(x_hbm, perm_hbm, out_hbm, idx_v, buf_v, sem):
        wid = lax.axis_index("s") * NC + lax.axis_index("c")
        b = wid // 2
        jbase = (wid % 2) * RPW
        row_off = b * S
        for c in range(NCHUNK):
            j0 = jbase + c * CHUNK
            pltpu.sync_copy(perm_hbm.at[pl.ds(j0, CHUNK)], idx_v)
            for i in range(CHUNK // L):
                sl = pl.ds(i * L, L)
                idx_v[sl] = idx_v[sl] + row_off
            pltpu.async_copy(x_hbm.at[idx_v], buf_v, sem).wait()
            pltpu.sync_copy(buf_v, out_hbm.at[pl.ds(row_off + j0, CHUNK)])

    return k


_sc_gather = _make_kernel()


def kernel(x, perm):
    x_flat = x.reshape(ROWS, D)
    out_flat = _sc_gather(x_flat, perm)
    return out_flat.reshape(B, S, D)
